# baseline (speedup 1.0000x reference)
import functools

import jax
import jax.numpy as jnp
import numpy as np
from jax import lax
from jax.experimental import pallas as pl
from jax.experimental.pallas import tpu as pltpu
from jax.experimental.pallas import tpu_sc as plsc

_DIM = 64
_NUM_REL = 500
_REL_PAD = 512

_NC = 2
_NS = 16
_NW = _NC * _NS
_GCHUNK = 128


@functools.lru_cache(maxsize=None)
def _gather_kernel(total_rows: int, dim: int):
    rows_per_w = total_rows // _NW
    n_chunks = rows_per_w // _GCHUNK
    mesh = plsc.VectorSubcoreMesh(core_axis_name="c", subcore_axis_name="s")

    @functools.partial(
        pl.kernel,
        mesh=mesh,
        out_type=jax.ShapeDtypeStruct((total_rows, dim), jnp.float32),
        scratch_types=[
            pltpu.VMEM((rows_per_w,), jnp.int32),
            pltpu.VMEM((rows_per_w, dim), jnp.float32),
            pltpu.SemaphoreType.DMA,
        ],
        compiler_params=pltpu.CompilerParams(use_tc_tiling_on_sc=False),
    )
    def gather(idx_hbm, table_hbm, out_hbm, idx_v, rows_v, sem):
        wid = lax.axis_index("s") * _NC + lax.axis_index("c")
        base = wid * rows_per_w
        pltpu.sync_copy(idx_hbm.at[pl.ds(base, rows_per_w)], idx_v)
        copies = [
            pltpu.async_copy(
                table_hbm.at[idx_v.at[pl.ds(j * _GCHUNK, _GCHUNK)]],
                rows_v.at[pl.ds(j * _GCHUNK, _GCHUNK), :],
                sem,
            )
            for j in range(n_chunks)
        ]
        for cp in copies:
            cp.wait()
        pltpu.sync_copy(rows_v, out_hbm.at[pl.ds(base, rows_per_w)])

    return gather


def _mlp_body(h_ref, t_ref, rel_ref, w1_ref, b1_ref, w2_ref, b2_ref, o_ref):
    r_avg = jnp.sum(rel_ref[...], axis=0, keepdims=True) * (1.0 / _NUM_REL)
    const = (
        jnp.dot(r_avg, w1_ref[_DIM : 2 * _DIM, :], preferred_element_type=jnp.float32)
        + b1_ref[...]
    )
    y = (
        jnp.dot(h_ref[...], w1_ref[0:_DIM, :], preferred_element_type=jnp.float32)
        + jnp.dot(t_ref[...], w1_ref[2 * _DIM : 3 * _DIM, :], preferred_element_type=jnp.float32)
        + const
    )
    y = y * 0.5 * (1.0 + lax.erf(y * np.float32(1.0 / np.sqrt(2.0))))
    o_ref[...] = jnp.dot(y, w2_ref[...], preferred_element_type=jnp.float32) + b2_ref[...]


def _mlp(gathered, relp, W1, b1_2d, W2, b2_2d, batch: int, block_b: int):
    grid = batch // block_b
    return pl.pallas_call(
        _mlp_body,
        grid=(grid,),
        in_specs=[
            pl.BlockSpec((block_b, _DIM), lambda i: (i, 0)),
            pl.BlockSpec((block_b, _DIM), lambda i, g=grid: (i + g, 0)),
            pl.BlockSpec((_REL_PAD, _DIM), lambda i: (0, 0)),
            pl.BlockSpec((3 * _DIM, _DIM), lambda i: (0, 0)),
            pl.BlockSpec((1, _DIM), lambda i: (0, 0)),
            pl.BlockSpec((_DIM, _NUM_REL), lambda i: (0, 0)),
            pl.BlockSpec((1, _NUM_REL), lambda i: (0, 0)),
        ],
        out_specs=pl.BlockSpec((block_b, _NUM_REL), lambda i: (i, 0)),
        out_shape=jax.ShapeDtypeStruct((batch, _NUM_REL), jnp.float32),
    )(gathered, gathered, relp, W1, b1_2d, W2, b2_2d)


def kernel(src, tgt, entity_emb, relation_emb, W1, b1, W2, b2):
    batch = src.shape[0]
    idx = jnp.concatenate([src.astype(jnp.int32), tgt.astype(jnp.int32)])
    gathered = _gather_kernel(2 * batch, _DIM)(idx, entity_emb)
    relp = jnp.zeros((_REL_PAD, _DIM), jnp.float32).at[:_NUM_REL].set(relation_emb)
    return _mlp(
        gathered,
        relp,
        W1,
        b1.reshape(1, _DIM),
        W2,
        b2.reshape(1, _NUM_REL),
        batch,
        block_b=2048,
    )
